# baseline (speedup 1.0000x reference)
import jax
import jax.numpy as jnp
from jax.experimental import pallas as pl

_EPS = 1e-12
_SEQ_BLOCK = 512
_BATCH_BLOCK = 2
_ROW_CHUNK = 64


def _ln_kernel(we_ref, pos_ref, w_ref, b_ref, out_ref):
    w = w_ref[...]
    b = b_ref[...]
    batch, s, hidden = we_ref.shape
    inv_h = 1.0 / hidden

    def body(i, _):
        r = i * _ROW_CHUNK
        for bi in range(batch):
            x = we_ref[bi, pl.ds(r, _ROW_CHUNK), :] + pos_ref[pl.ds(r, _ROW_CHUNK), :]
            s1 = jnp.sum(x, axis=-1, keepdims=True)
            s2 = jnp.sum(x * x, axis=-1, keepdims=True)
            mean = s1 * inv_h
            var = s2 * inv_h - mean * mean
            rs = jax.lax.rsqrt(var + _EPS)
            out_ref[bi, pl.ds(r, _ROW_CHUNK), :] = (
                (x - mean) * (rs * w) + b)
        return 0

    jax.lax.fori_loop(0, s // _ROW_CHUNK, body, 0)


def kernel(word_embeddings, pos_table, ln_weight, ln_bias):
    batch, seq, hidden = word_embeddings.shape
    s = _SEQ_BLOCK
    bb = _BATCH_BLOCK
    grid = (seq // s, batch // bb)
    return pl.pallas_call(
        _ln_kernel,
        grid=grid,
        in_specs=[
            pl.BlockSpec((bb, s, hidden), lambda i, j: (j, i, 0)),
            pl.BlockSpec((s, hidden), lambda i, j: (i, 0)),
            pl.BlockSpec((hidden,), lambda i, j: (0,)),
            pl.BlockSpec((hidden,), lambda i, j: (0,)),
        ],
        out_specs=pl.BlockSpec((bb, s, hidden), lambda i, j: (j, i, 0)),
        out_shape=jax.ShapeDtypeStruct((batch, seq, hidden), jnp.float32),
    )(word_embeddings, pos_table[:seq], ln_weight, ln_bias)
